# Initial kernel scaffold; baseline (speedup 1.0000x reference)
import jax
import jax.numpy as jnp
from jax.experimental import pallas as pl


def kernel(scores, pos_type_ids, neg_type_ids, levels, margin_ratio, margins):
    raise NotImplementedError("write your pallas kernel here")


# speedup vs baseline: 1.9874x; 1.9874x over previous
import functools

import numpy as np
import jax
import jax.numpy as jnp
from jax import lax
from jax.experimental import pallas as pl
from jax.experimental.pallas import tpu as pltpu
from jax.experimental.pallas import tpu_sc as plsc

B, T, P, N = 4096, 1000, 20, 50
PN = P * N
NC, NS = 2, 16
NW = NC * NS
ROWS_W = B // NW
RB = 16
NBLK = ROWS_W // RB
CHUNK = RB * PN
MB = RB * P

_j = np.arange(CHUNK)
_ROFF = ((_j // PN) * T).astype(np.int32)
_MIDX = ((_j // PN) * P + (_j % PN) // N).astype(np.int32)
_j2 = np.arange(MB)
_RMOFF = ((_j2 // P) * T).astype(np.int32)

_mesh = plsc.VectorSubcoreMesh(core_axis_name="c", subcore_axis_name="s")


@functools.partial(
    pl.kernel,
    out_type=jax.ShapeDtypeStruct((NW, 32), jnp.float32),
    mesh=_mesh,
    compiler_params=pltpu.CompilerParams(needs_layout_passes=False),
    scratch_types=[
        pltpu.VMEM((RB * T,), jnp.float32),
        pltpu.VMEM((CHUNK,), jnp.int32),
        pltpu.VMEM((MB,), jnp.int32),
        pltpu.VMEM((MB,), jnp.int32),
        pltpu.VMEM((MB,), jnp.float32),
        pltpu.VMEM((16,), jnp.float32),
        pltpu.VMEM((CHUNK,), jnp.int32),
        pltpu.VMEM((CHUNK,), jnp.int32),
        pltpu.VMEM((MB,), jnp.int32),
        pltpu.VMEM((32,), jnp.float32),
    ],
)
def _hinge_sc(scores_hbm, neg_hbm, pos_hbm, lev_hbm, marg_hbm,
              roff_hbm, midx_hbm, rmoff_hbm, out_hbm,
              scores_v, neg_v, pos_v, lev_v, m_v, marg_v,
              roff_v, midx_v, rmoff_v, out_v):
    wid = lax.axis_index("s") * NC + lax.axis_index("c")
    pltpu.sync_copy(marg_hbm, marg_v)
    pltpu.sync_copy(roff_hbm, roff_v)
    pltpu.sync_copy(midx_hbm, midx_v)
    pltpu.sync_copy(rmoff_hbm, rmoff_v)
    row0w = wid * ROWS_W

    def block(bk, carry):
        acc, cnt = carry
        row0 = row0w + bk * RB
        pltpu.sync_copy(scores_hbm.at[pl.ds(row0 * T, RB * T)], scores_v)
        pltpu.sync_copy(neg_hbm.at[pl.ds(row0 * PN, CHUNK)], neg_v)
        pltpu.sync_copy(pos_hbm.at[pl.ds(row0 * P, MB)], pos_v)
        pltpu.sync_copy(lev_hbm.at[pl.ds(row0 * P, MB)], lev_v)

        def mloop(i, t):
            off = i * 16
            pos = pos_v[pl.ds(off, 16)]
            lev = lev_v[pl.ds(off, 16)]
            safe_pos = jnp.where(pos == -1, 0, pos)
            pidx = rmoff_v[pl.ds(off, 16)] + safe_pos
            psc = plsc.load_gather(scores_v, [pidx])
            mg = plsc.load_gather(marg_v, [lev])
            m_v[pl.ds(off, 16)] = mg - psc
            return t
        lax.fori_loop(0, MB // 16, mloop, 0)

        def nloop(i, c):
            a, ct = c
            off = i * 16
            neg = neg_v[pl.ds(off, 16)]
            msk = neg != -1
            safe_neg = jnp.where(msk, neg, 0)
            gidx = roff_v[pl.ds(off, 16)] + safe_neg
            nsc = plsc.load_gather(scores_v, [gidx])
            mval = plsc.load_gather(m_v, [midx_v[pl.ds(off, 16)]])
            v = jnp.maximum(mval + nsc, 0.0)
            a = a + jnp.where(msk, v, 0.0)
            ct = ct + jnp.where(msk, 1.0, 0.0)
            return (a, ct)
        return lax.fori_loop(0, CHUNK // 16, nloop, (acc, cnt))

    zero = jnp.zeros((16,), jnp.float32)
    acc, cnt = lax.fori_loop(0, NBLK, block, (zero, zero))
    out_v[pl.ds(0, 16)] = acc
    out_v[pl.ds(16, 16)] = cnt
    pltpu.sync_copy(out_v, out_hbm.at[wid])


def kernel(scores, pos_type_ids, neg_type_ids, levels, margin_ratio, margins):
    marg = (margins * margin_ratio).astype(jnp.float32)
    parts = _hinge_sc(
        scores.reshape(-1),
        neg_type_ids.reshape(-1),
        pos_type_ids.reshape(-1),
        levels.reshape(-1),
        marg,
        jnp.asarray(_ROFF),
        jnp.asarray(_MIDX),
        jnp.asarray(_RMOFF),
    )
    pr = parts.reshape(NW, 2, 16)
    return pr[:, 0].sum() / pr[:, 1].sum()


# speedup vs baseline: 2.1224x; 1.0679x over previous
import functools

import numpy as np
import jax
import jax.numpy as jnp
from jax import lax
from jax.experimental import pallas as pl
from jax.experimental.pallas import tpu as pltpu
from jax.experimental.pallas import tpu_sc as plsc

B, T, P, N = 4096, 1000, 20, 50
PN = P * N
NC, NS = 2, 16
NW = NC * NS
ROWS_W = B // NW
RB = 16
NBLK = ROWS_W // RB
CHUNK = RB * PN
MB = RB * P

_j = np.arange(CHUNK)
_ROFF = ((_j // PN) * T).astype(np.int32)
_MIDX = ((_j // PN) * P + (_j % PN) // N).astype(np.int32)
_j2 = np.arange(MB)
_RMOFF = ((_j2 // P) * T).astype(np.int32)

_mesh = plsc.VectorSubcoreMesh(core_axis_name="c", subcore_axis_name="s")


@functools.partial(
    pl.kernel,
    out_type=jax.ShapeDtypeStruct((NW, 32), jnp.float32),
    mesh=_mesh,
    compiler_params=pltpu.CompilerParams(needs_layout_passes=False),
    scratch_types=[
        pltpu.VMEM((RB * T,), jnp.float32),
        pltpu.VMEM((CHUNK,), jnp.int32),
        pltpu.VMEM((MB,), jnp.int32),
        pltpu.VMEM((MB,), jnp.int32),
        pltpu.VMEM((MB,), jnp.float32),
        pltpu.VMEM((16,), jnp.float32),
        pltpu.VMEM((CHUNK,), jnp.int32),
        pltpu.VMEM((CHUNK,), jnp.int32),
        pltpu.VMEM((MB,), jnp.int32),
        pltpu.VMEM((32,), jnp.float32),
    ],
)
def _hinge_sc(scores_hbm, neg_hbm, pos_hbm, lev_hbm, marg_hbm,
              roff_hbm, midx_hbm, rmoff_hbm, out_hbm,
              scores_v, neg_v, pos_v, lev_v, m_v, marg_v,
              roff_v, midx_v, rmoff_v, out_v):
    wid = lax.axis_index("s") * NC + lax.axis_index("c")
    pltpu.sync_copy(marg_hbm, marg_v)
    pltpu.sync_copy(roff_hbm, roff_v)
    pltpu.sync_copy(midx_hbm, midx_v)
    pltpu.sync_copy(rmoff_hbm, rmoff_v)
    row0w = wid * ROWS_W

    def block(bk, carry):
        acc, cnt = carry
        row0 = row0w + bk * RB
        pltpu.sync_copy(scores_hbm.at[pl.ds(row0 * T, RB * T)], scores_v)
        pltpu.sync_copy(neg_hbm.at[pl.ds(row0 * PN, CHUNK)], neg_v)
        pltpu.sync_copy(pos_hbm.at[pl.ds(row0 * P, MB)], pos_v)
        pltpu.sync_copy(lev_hbm.at[pl.ds(row0 * P, MB)], lev_v)

        @plsc.parallel_loop(0, MB // 16, unroll=4)
        def mloop(i):
            off = i * 16
            pos = pos_v[pl.ds(off, 16)]
            lev = lev_v[pl.ds(off, 16)]
            safe_pos = jnp.where(pos == -1, 0, pos)
            pidx = rmoff_v[pl.ds(off, 16)] + safe_pos
            psc = plsc.load_gather(scores_v, [pidx])
            mg = plsc.load_gather(marg_v, [lev])
            m_v[pl.ds(off, 16)] = mg - psc

        @plsc.parallel_loop(0, CHUNK // 16, unroll=8, carry=(acc, cnt))
        def nloop(i, c):
            a, ct = c
            off = i * 16
            neg = neg_v[pl.ds(off, 16)]
            msk = neg != -1
            safe_neg = jnp.where(msk, neg, 0)
            gidx = roff_v[pl.ds(off, 16)] + safe_neg
            nsc = plsc.load_gather(scores_v, [gidx])
            mval = plsc.load_gather(m_v, [midx_v[pl.ds(off, 16)]])
            v = jnp.maximum(mval + nsc, 0.0)
            a = a + jnp.where(msk, v, 0.0)
            ct = ct + jnp.where(msk, 1.0, 0.0)
            return (a, ct)
        return nloop

    zero = jnp.zeros((16,), jnp.float32)
    acc, cnt = lax.fori_loop(0, NBLK, block, (zero, zero))
    out_v[pl.ds(0, 16)] = acc
    out_v[pl.ds(16, 16)] = cnt
    pltpu.sync_copy(out_v, out_hbm.at[wid])


def kernel(scores, pos_type_ids, neg_type_ids, levels, margin_ratio, margins):
    marg = (margins * margin_ratio).astype(jnp.float32)
    parts = _hinge_sc(
        scores.reshape(-1),
        neg_type_ids.reshape(-1),
        pos_type_ids.reshape(-1),
        levels.reshape(-1),
        marg,
        jnp.asarray(_ROFF),
        jnp.asarray(_MIDX),
        jnp.asarray(_RMOFF),
    )
    pr = parts.reshape(NW, 2, 16)
    return pr[:, 0].sum() / pr[:, 1].sum()


# speedup vs baseline: 2.2189x; 1.0455x over previous
import functools

import numpy as np
import jax
import jax.numpy as jnp
from jax import lax
from jax.experimental import pallas as pl
from jax.experimental.pallas import tpu as pltpu
from jax.experimental.pallas import tpu_sc as plsc

B, T, P, N = 4096, 1000, 20, 50
PN = P * N
NC, NS = 2, 16
NW = NC * NS
ROWS_W = B // NW
RB = 16
NBLK = ROWS_W // RB
CHUNK = RB * PN
MB = RB * P

_j = np.arange(CHUNK)
_RIDX = (_j // PN).astype(np.int32)
_MIDX = ((_j // PN) * P + (_j % PN) // N).astype(np.int32)
_j2 = np.arange(MB)
_RMIDX = (_j2 // P).astype(np.int32)

_mesh = plsc.VectorSubcoreMesh(core_axis_name="c", subcore_axis_name="s")


@functools.partial(
    pl.kernel,
    out_type=jax.ShapeDtypeStruct((NW, 32), jnp.float32),
    mesh=_mesh,
    compiler_params=pltpu.CompilerParams(needs_layout_passes=False),
    scratch_types=[
        pltpu.VMEM((RB, T), jnp.float32),
        pltpu.VMEM((CHUNK,), jnp.int32),
        pltpu.VMEM((MB,), jnp.int32),
        pltpu.VMEM((MB,), jnp.int32),
        pltpu.VMEM((MB,), jnp.float32),
        pltpu.VMEM((16,), jnp.float32),
        pltpu.VMEM((CHUNK,), jnp.int32),
        pltpu.VMEM((CHUNK,), jnp.int32),
        pltpu.VMEM((MB,), jnp.int32),
        pltpu.VMEM((32,), jnp.float32),
    ],
)
def _hinge_sc(scores_hbm, neg_hbm, pos_hbm, lev_hbm, marg_hbm,
              ridx_hbm, midx_hbm, rmidx_hbm, out_hbm,
              scores_v, neg_v, pos_v, lev_v, m_v, marg_v,
              ridx_v, midx_v, rmidx_v, out_v):
    wid = lax.axis_index("s") * NC + lax.axis_index("c")
    pltpu.sync_copy(marg_hbm, marg_v)
    pltpu.sync_copy(ridx_hbm, ridx_v)
    pltpu.sync_copy(midx_hbm, midx_v)
    pltpu.sync_copy(rmidx_hbm, rmidx_v)
    row0w = wid * ROWS_W

    def block(bk, carry):
        acc, cnt = carry
        row0 = row0w + bk * RB
        pltpu.sync_copy(scores_hbm.at[pl.ds(row0, RB)], scores_v)
        pltpu.sync_copy(neg_hbm.at[pl.ds(row0 * PN, CHUNK)], neg_v)
        pltpu.sync_copy(pos_hbm.at[pl.ds(row0 * P, MB)], pos_v)
        pltpu.sync_copy(lev_hbm.at[pl.ds(row0 * P, MB)], lev_v)

        @plsc.parallel_loop(0, MB // 16, unroll=4)
        def mloop(i):
            off = i * 16
            pos = pos_v[pl.ds(off, 16)]
            lev = lev_v[pl.ds(off, 16)]
            safe_pos = jnp.where(pos == -1, 0, pos)
            psc = plsc.load_gather(scores_v, [rmidx_v[pl.ds(off, 16)], safe_pos])
            mg = plsc.load_gather(marg_v, [lev])
            m_v[pl.ds(off, 16)] = mg - psc

        @plsc.parallel_loop(0, CHUNK // 16, unroll=8, carry=(acc, cnt))
        def nloop(i, c):
            a, ct = c
            off = i * 16
            neg = neg_v[pl.ds(off, 16)]
            msk = neg != -1
            safe_neg = jnp.where(msk, neg, 0)
            nsc = plsc.load_gather(scores_v, [ridx_v[pl.ds(off, 16)], safe_neg])
            mval = plsc.load_gather(m_v, [midx_v[pl.ds(off, 16)]])
            v = jnp.maximum(mval + nsc, 0.0)
            a = a + jnp.where(msk, v, 0.0)
            ct = ct + jnp.where(msk, 1.0, 0.0)
            return (a, ct)
        return nloop

    zero = jnp.zeros((16,), jnp.float32)
    acc, cnt = lax.fori_loop(0, NBLK, block, (zero, zero))
    out_v[pl.ds(0, 16)] = acc
    out_v[pl.ds(16, 16)] = cnt
    pltpu.sync_copy(out_v, out_hbm.at[wid])


def kernel(scores, pos_type_ids, neg_type_ids, levels, margin_ratio, margins):
    marg = (margins * margin_ratio).astype(jnp.float32)
    parts = _hinge_sc(
        scores,
        neg_type_ids.reshape(-1),
        pos_type_ids.reshape(-1),
        levels.reshape(-1),
        marg,
        jnp.asarray(_RIDX),
        jnp.asarray(_MIDX),
        jnp.asarray(_RMIDX),
    )
    pr = parts.reshape(NW, 2, 16)
    return pr[:, 0].sum() / pr[:, 1].sum()


# speedup vs baseline: 2.6860x; 1.2105x over previous
import functools

import numpy as np
import jax
import jax.numpy as jnp
from jax import lax
from jax.experimental import pallas as pl
from jax.experimental.pallas import tpu as pltpu
from jax.experimental.pallas import tpu_sc as plsc

B, T, P, N = 4096, 1000, 20, 50
PN = P * N
NC, NS = 2, 16
NW = NC * NS
ROWS_W = B // NW
RB = 16
NBLK = ROWS_W // RB
MB = RB * P
NFULL = PN // 16
TAIL0 = PN - 16
NTAILV = PN - NFULL * 16

_jc = np.arange(PN)
_MCOL = (_jc // N).astype(np.int32)
_j2 = np.arange(MB)
_MRI = (_j2 // P).astype(np.int32)
_MPI = (_j2 % P).astype(np.int32)

_mesh = plsc.VectorSubcoreMesh(core_axis_name="c", subcore_axis_name="s")


@functools.partial(
    pl.kernel,
    out_type=jax.ShapeDtypeStruct((NW, 32), jnp.float32),
    mesh=_mesh,
    compiler_params=pltpu.CompilerParams(needs_layout_passes=False),
    scratch_types=[
        pltpu.VMEM((RB, T), jnp.float32),
        pltpu.VMEM((RB, T), jnp.float32),
        pltpu.VMEM((RB, PN), jnp.int32),
        pltpu.VMEM((RB, PN), jnp.int32),
        pltpu.VMEM((MB,), jnp.int32),
        pltpu.VMEM((MB,), jnp.int32),
        pltpu.VMEM((MB,), jnp.int32),
        pltpu.VMEM((MB,), jnp.int32),
        pltpu.VMEM((MB,), jnp.float32),
        pltpu.VMEM((16,), jnp.float32),
        pltpu.VMEM((PN,), jnp.int32),
        pltpu.VMEM((MB,), jnp.int32),
        pltpu.VMEM((MB,), jnp.int32),
        pltpu.VMEM((32,), jnp.float32),
        pltpu.SemaphoreType.DMA,
        pltpu.SemaphoreType.DMA,
    ],
)
def _hinge_sc(scores_hbm, neg2d_hbm, pos_hbm, lev_hbm, marg_hbm,
              mcol_hbm, mri_hbm, mpi_hbm, out_hbm,
              sc0, sc1, ng0, ng1, po0, po1, lv0, lv1,
              m_v, marg_v, mcol_v, mri_v, mpi_v, out_v, sem0, sem1):
    wid = lax.axis_index("s") * NC + lax.axis_index("c")
    pltpu.sync_copy(marg_hbm, marg_v)
    pltpu.sync_copy(mcol_hbm, mcol_v)
    pltpu.sync_copy(mri_hbm, mri_v)
    pltpu.sync_copy(mpi_hbm, mpi_v)
    row0w = wid * ROWS_W
    iota16 = lax.iota(jnp.int32, 16)
    bufs = ((sc0, ng0, po0, lv0, sem0), (sc1, ng1, po1, lv1, sem1))
    tail_keep = lax.iota(jnp.int32, 16) >= (16 - NTAILV)

    def issue(bk):
        sc, ng, po, lv, sem = bufs[bk % 2]
        row0 = row0w + bk * RB
        return (
            pltpu.async_copy(scores_hbm.at[pl.ds(row0, RB)], sc, sem),
            pltpu.async_copy(neg2d_hbm.at[pl.ds(row0, RB)], ng, sem),
            pltpu.async_copy(pos_hbm.at[pl.ds(row0 * P, MB)], po, sem),
            pltpu.async_copy(lev_hbm.at[pl.ds(row0 * P, MB)], lv, sem),
        )

    pending = issue(0)
    acc = jnp.zeros((16,), jnp.float32)
    cnt = jnp.zeros((16,), jnp.float32)

    for bk in range(NBLK):
        sc, ng, po, lv, _ = bufs[bk % 2]
        for h in pending:
            h.wait()
        if bk + 1 < NBLK:
            pending = issue(bk + 1)

        @plsc.parallel_loop(0, MB // 16, unroll=4)
        def mloop(i):
            off = i * 16
            mri = mri_v[pl.ds(off, 16)]
            pos = po[pl.ds(off, 16)]
            lev = lv[pl.ds(off, 16)]
            safe_pos = jnp.where(pos == -1, 0, pos)
            psc = plsc.load_gather(sc, [mri, safe_pos])
            mg = plsc.load_gather(marg_v, [lev])
            m_v[pl.ds(off, 16)] = mg - psc

        def chunk(rsplat, rp, off, extra_mask, a, ct):
            mcolc = mcol_v[pl.ds(off, 16)]
            mi = mcolc + rp
            jv = iota16 + jnp.full((16,), off, jnp.int32)
            neg = plsc.load_gather(ng, [rsplat, jv])
            msk = neg != -1
            if extra_mask is not None:
                msk = msk & extra_mask
            safe_neg = jnp.where(msk, neg, 0)
            nsc = plsc.load_gather(sc, [rsplat, safe_neg])
            mval = plsc.load_gather(m_v, [mi])
            v = jnp.maximum(mval + nsc, 0.0)
            a = a + jnp.where(msk, v, 0.0)
            ct = ct + jnp.where(msk, 1.0, 0.0)
            return a, ct

        def rowloop(r, carry):
            a, ct = carry
            rsplat = jnp.full((16,), r, jnp.int32)
            rp = rsplat * P

            @plsc.parallel_loop(0, NFULL, unroll=4, carry=(a, ct))
            def nloop(i, c):
                ai, ci = c
                return chunk(rsplat, rp, i * 16, None, ai, ci)

            a, ct = nloop
            return chunk(rsplat, rp, TAIL0, tail_keep, a, ct)

        acc, cnt = lax.fori_loop(0, RB, rowloop, (acc, cnt))

    out_v[pl.ds(0, 16)] = acc
    out_v[pl.ds(16, 16)] = cnt
    pltpu.sync_copy(out_v, out_hbm.at[wid])


def kernel(scores, pos_type_ids, neg_type_ids, levels, margin_ratio, margins):
    marg = (margins * margin_ratio).astype(jnp.float32)
    parts = _hinge_sc(
        scores,
        neg_type_ids.reshape(B, PN),
        pos_type_ids.reshape(-1),
        levels.reshape(-1),
        marg,
        jnp.asarray(_MCOL),
        jnp.asarray(_MRI),
        jnp.asarray(_MPI),
    )
    pr = parts.reshape(NW, 2, 16)
    return pr[:, 0].sum() / pr[:, 1].sum()


# speedup vs baseline: 2.8466x; 1.0598x over previous
import functools

import numpy as np
import jax
import jax.numpy as jnp
from jax import lax
from jax.experimental import pallas as pl
from jax.experimental.pallas import tpu as pltpu
from jax.experimental.pallas import tpu_sc as plsc

B, T, P, N = 4096, 1000, 20, 50
PN = P * N
NC, NS = 2, 16
NW = NC * NS
ROWS_W = B // NW
RB = 16
NBLK = ROWS_W // RB
MB = RB * P
NFULL = PN // 16
TAIL0 = PN - 16
NTAILV = PN - NFULL * 16

_jc = np.arange(PN)
_MCOL = (_jc // N).astype(np.int32)
_j2 = np.arange(MB)
_MRI = (_j2 // P).astype(np.int32)
_MPI = (_j2 % P).astype(np.int32)

_mesh = plsc.VectorSubcoreMesh(core_axis_name="c", subcore_axis_name="s")


@functools.partial(
    pl.kernel,
    out_type=jax.ShapeDtypeStruct((NW, 32), jnp.float32),
    mesh=_mesh,
    compiler_params=pltpu.CompilerParams(needs_layout_passes=False),
    scratch_types=[
        pltpu.VMEM((RB, T), jnp.float32),
        pltpu.VMEM((RB, T), jnp.float32),
        pltpu.VMEM((RB, PN), jnp.int32),
        pltpu.VMEM((RB, PN), jnp.int32),
        pltpu.VMEM((MB,), jnp.int32),
        pltpu.VMEM((MB,), jnp.int32),
        pltpu.VMEM((MB,), jnp.int32),
        pltpu.VMEM((MB,), jnp.int32),
        pltpu.VMEM((MB,), jnp.float32),
        pltpu.VMEM((16,), jnp.float32),
        pltpu.VMEM((PN,), jnp.int32),
        pltpu.VMEM((MB,), jnp.int32),
        pltpu.VMEM((MB,), jnp.int32),
        pltpu.VMEM((32,), jnp.float32),
        pltpu.SemaphoreType.DMA,
        pltpu.SemaphoreType.DMA,
    ],
)
def _hinge_sc(scores_hbm, neg2d_hbm, pos_hbm, lev_hbm, marg_hbm,
              mcol_hbm, mri_hbm, mpi_hbm, out_hbm,
              sc0, sc1, ng0, ng1, po0, po1, lv0, lv1,
              m_v, marg_v, mcol_v, mri_v, mpi_v, out_v, sem0, sem1):
    wid = lax.axis_index("s") * NC + lax.axis_index("c")
    pltpu.sync_copy(marg_hbm, marg_v)
    pltpu.sync_copy(mcol_hbm, mcol_v)
    pltpu.sync_copy(mri_hbm, mri_v)
    pltpu.sync_copy(mpi_hbm, mpi_v)
    row0w = wid * ROWS_W
    iota16 = lax.iota(jnp.int32, 16)
    bufs = ((sc0, ng0, po0, lv0, sem0), (sc1, ng1, po1, lv1, sem1))
    tail_keep = lax.iota(jnp.int32, 16) >= (16 - NTAILV)

    def issue(bk):
        sc, ng, po, lv, sem = bufs[bk % 2]
        row0 = row0w + bk * RB
        return (
            pltpu.async_copy(scores_hbm.at[pl.ds(row0, RB)], sc, sem),
            pltpu.async_copy(neg2d_hbm.at[pl.ds(row0, RB)], ng, sem),
            pltpu.async_copy(pos_hbm.at[pl.ds(row0 * P, MB)], po, sem),
            pltpu.async_copy(lev_hbm.at[pl.ds(row0 * P, MB)], lv, sem),
        )

    pending = issue(0)
    acc = (jnp.zeros((16,), jnp.float32), jnp.zeros((16,), jnp.float32),
           jnp.zeros((16,), jnp.int32), jnp.zeros((16,), jnp.int32))

    for bk in range(NBLK):
        sc, ng, po, lv, _ = bufs[bk % 2]
        for h in pending:
            h.wait()
        if bk + 1 < NBLK:
            pending = issue(bk + 1)

        @plsc.parallel_loop(0, MB // 16, unroll=4)
        def mloop(i):
            off = i * 16
            mri = mri_v[pl.ds(off, 16)]
            pos = po[pl.ds(off, 16)]
            lev = lv[pl.ds(off, 16)]
            safe_pos = jnp.where(pos == -1, 0, pos)
            psc = plsc.load_gather(sc, [mri, safe_pos])
            mg = plsc.load_gather(marg_v, [lev])
            m_v[pl.ds(off, 16)] = mg - psc

        def chunk(rsplat, rp, off, extra_mask, a, ct):
            mcolc = mcol_v[pl.ds(off, 16)]
            mi = mcolc + rp
            jv = iota16 + jnp.full((16,), off, jnp.int32)
            neg = plsc.load_gather(ng, [rsplat, jv])
            msk = neg != -1
            if extra_mask is not None:
                msk = msk & extra_mask
            safe_neg = jnp.where(msk, neg, 0)
            nsc = plsc.load_gather(sc, [rsplat, safe_neg])
            mval = plsc.load_gather(m_v, [mi])
            v = jnp.maximum(mval + nsc, 0.0)
            a = a + jnp.where(msk, v, 0.0)
            ct = ct + jnp.where(msk, 1, 0)
            return a, ct

        def rowloop(r, carry):
            a0, a1, c0, c1 = carry
            rsplat = jnp.full((16,), r, jnp.int32)
            rp = rsplat * P

            @plsc.parallel_loop(0, NFULL // 2, unroll=4, carry=(a0, a1, c0, c1))
            def nloop(i, c):
                ai0, ai1, ci0, ci1 = c
                off = i * 32
                ai0, ci0 = chunk(rsplat, rp, off, None, ai0, ci0)
                ai1, ci1 = chunk(rsplat, rp, off + 16, None, ai1, ci1)
                return (ai0, ai1, ci0, ci1)

            a0, a1, c0, c1 = nloop
            a0, c0 = chunk(rsplat, rp, TAIL0, tail_keep, a0, c0)
            return (a0, a1, c0, c1)

        acc = lax.fori_loop(0, RB, rowloop, acc)

    out_v[pl.ds(0, 16)] = acc[0] + acc[1]
    out_v[pl.ds(16, 16)] = (acc[2] + acc[3]).astype(jnp.float32)
    pltpu.sync_copy(out_v, out_hbm.at[wid])


def kernel(scores, pos_type_ids, neg_type_ids, levels, margin_ratio, margins):
    marg = (margins * margin_ratio).astype(jnp.float32)
    parts = _hinge_sc(
        scores,
        neg_type_ids.reshape(B, PN),
        pos_type_ids.reshape(-1),
        levels.reshape(-1),
        marg,
        jnp.asarray(_MCOL),
        jnp.asarray(_MRI),
        jnp.asarray(_MPI),
    )
    pr = parts.reshape(NW, 2, 16)
    return pr[:, 0].sum() / pr[:, 1].sum()


# speedup vs baseline: 3.1922x; 1.1214x over previous
import functools

import numpy as np
import jax
import jax.numpy as jnp
from jax import lax
from jax.experimental import pallas as pl
from jax.experimental.pallas import tpu as pltpu
from jax.experimental.pallas import tpu_sc as plsc

B, T, P, N = 4096, 1000, 20, 50
PN = P * N
NC, NS = 2, 16
NW = NC * NS
ROWS_W = B // NW
RB = 16
NBLK = ROWS_W // RB
MB = RB * P
NFULL = PN // 16
TAIL0 = PN - 16
NTAILV = PN - NFULL * 16

_j2 = np.arange(MB)
_MRI = (_j2 // P).astype(np.int32)

_mesh = plsc.VectorSubcoreMesh(core_axis_name="c", subcore_axis_name="s")


@functools.partial(
    pl.kernel,
    out_type=jax.ShapeDtypeStruct((NW, 64), jnp.float32),
    mesh=_mesh,
    compiler_params=pltpu.CompilerParams(needs_layout_passes=False),
    scratch_types=[
        pltpu.VMEM((RB, T), jnp.float32),
        pltpu.VMEM((RB, T), jnp.float32),
        pltpu.VMEM((RB, PN), jnp.int32),
        pltpu.VMEM((RB, PN), jnp.int32),
        pltpu.VMEM((MB,), jnp.int32),
        pltpu.VMEM((MB,), jnp.int32),
        pltpu.VMEM((MB,), jnp.int32),
        pltpu.VMEM((MB,), jnp.int32),
        pltpu.VMEM((MB,), jnp.float32),
        pltpu.VMEM((16,), jnp.float32),
        pltpu.VMEM((MB,), jnp.int32),
        pltpu.VMEM((64,), jnp.float32),
        pltpu.SemaphoreType.DMA,
        pltpu.SemaphoreType.DMA,
    ],
)
def _hinge_sc(scores_hbm, neg2d_hbm, pos_hbm, lev_hbm, marg_hbm,
              mri_hbm, out_hbm,
              sc0, sc1, ng0, ng1, po0, po1, lv0, lv1,
              m_v, marg_v, mri_v, out_v, sem0, sem1):
    wid = lax.axis_index("s") * NC + lax.axis_index("c")
    pltpu.sync_copy(marg_hbm, marg_v)
    pltpu.sync_copy(mri_hbm, mri_v)
    row0w = wid * ROWS_W
    iota16 = lax.iota(jnp.int32, 16)
    bufs = ((sc0, ng0, po0, lv0, sem0), (sc1, ng1, po1, lv1, sem1))
    tail_keep = iota16 >= (16 - NTAILV)

    def issue(slot, bk):
        sc, ng, po, lv, sem = bufs[slot]
        row0 = row0w + bk * RB
        pltpu.async_copy(scores_hbm.at[pl.ds(row0, RB)], sc, sem)
        pltpu.async_copy(neg2d_hbm.at[pl.ds(row0, RB)], ng, sem)
        pltpu.async_copy(pos_hbm.at[pl.ds(row0 * P, MB)], po, sem)
        pltpu.async_copy(lev_hbm.at[pl.ds(row0 * P, MB)], lv, sem)

    def wait_slot(slot):
        sc, ng, po, lv, sem = bufs[slot]
        pltpu.make_async_copy(scores_hbm.at[pl.ds(0, RB)], sc, sem).wait()
        pltpu.make_async_copy(neg2d_hbm.at[pl.ds(0, RB)], ng, sem).wait()
        pltpu.make_async_copy(pos_hbm.at[pl.ds(0, MB)], po, sem).wait()
        pltpu.make_async_copy(lev_hbm.at[pl.ds(0, MB)], lv, sem).wait()

    def compute_block(slot, carry):
        sc, ng, po, lv, _ = bufs[slot]

        @plsc.parallel_loop(0, MB // 16, unroll=4)
        def mloop(i):
            off = i * 16
            mri = mri_v[pl.ds(off, 16)]
            pos = po[pl.ds(off, 16)]
            lev = lv[pl.ds(off, 16)]
            safe_pos = jnp.where(pos == -1, 0, pos)
            psc = plsc.load_gather(sc, [mri, safe_pos])
            mg = plsc.load_gather(marg_v, [lev])
            m_v[pl.ds(off, 16)] = mg - psc

        def rowloop(r, cr):
            a = list(cr)
            rsplat = jnp.full((16,), r, jnp.int32)
            rp = r * P
            msp = [plsc.load_gather(m_v, [jnp.full((16,), rp + p, jnp.int32)])
                   for p in range(P)]

            def chunk(k, off, msel, extra_mask, a):
                neg = ng[r, pl.ds(off, 16)]
                msk = neg != -1
                if extra_mask is not None:
                    msk = msk & extra_mask
                safe_neg = jnp.where(msk, neg, 0)
                nsc = plsc.load_gather(sc, [rsplat, safe_neg])
                v = jnp.maximum(msel + nsc, 0.0)
                s = k % 4
                a[s] = a[s] + jnp.where(msk, v, 0.0)
                a[4 + s] = a[4 + s] + jnp.where(msk, 1, 0)
                return a

            for k in range(NFULL):
                off = k * 16
                p_lo = off // N
                p_hi = (off + 15) // N
                if p_lo == p_hi:
                    msel = msp[p_lo]
                else:
                    b = N * p_hi - off
                    msel = jnp.where(iota16 >= b, msp[p_hi], msp[p_lo])
                a = chunk(k, off, msel, None, a)
            a = chunk(NFULL, TAIL0, msp[P - 1], tail_keep, a)
            return tuple(a)

        return lax.fori_loop(0, RB, rowloop, carry)

    issue(0, 0)
    issue(1, 1)
    zf = jnp.zeros((16,), jnp.float32)
    zi = jnp.zeros((16,), jnp.int32)
    acc = (zf, zf, zf, zf, zi, zi, zi, zi)

    def blockpair(t, carry):
        wait_slot(0)
        carry = compute_block(0, carry)

        @pl.when(t < NBLK // 2 - 1)
        def _():
            issue(0, 2 * t + 2)

        wait_slot(1)
        carry = compute_block(1, carry)

        @pl.when(t < NBLK // 2 - 1)
        def _():
            issue(1, 2 * t + 3)

        return carry

    acc = lax.fori_loop(0, NBLK // 2, blockpair, acc)

    out_v[pl.ds(0, 16)] = acc[0] + acc[1]
    out_v[pl.ds(16, 16)] = acc[2] + acc[3]
    out_v[pl.ds(32, 16)] = (acc[4] + acc[5]).astype(jnp.float32)
    out_v[pl.ds(48, 16)] = (acc[6] + acc[7]).astype(jnp.float32)
    pltpu.sync_copy(out_v, out_hbm.at[wid])


def kernel(scores, pos_type_ids, neg_type_ids, levels, margin_ratio, margins):
    marg = (margins * margin_ratio).astype(jnp.float32)
    parts = _hinge_sc(
        scores,
        neg_type_ids.reshape(B, PN),
        pos_type_ids.reshape(-1),
        levels.reshape(-1),
        marg,
        jnp.asarray(_MRI),
    )
    pr = parts.reshape(NW, 2, 32)
    return pr[:, 0].sum() / pr[:, 1].sum()


# speedup vs baseline: 3.2858x; 1.0293x over previous
import functools

import numpy as np
import jax
import jax.numpy as jnp
from jax import lax
from jax.experimental import pallas as pl
from jax.experimental.pallas import tpu as pltpu
from jax.experimental.pallas import tpu_sc as plsc

B, T, P, N = 4096, 1000, 20, 50
PN = P * N
NC, NS = 2, 16
NW = NC * NS
ROWS_W = B // NW
RB = 16
NBLK = ROWS_W // RB
MB = RB * P
NFULL = PN // 16
TAIL0 = PN - 16
NTAILV = PN - NFULL * 16

_j2 = np.arange(MB)
_MRI = (_j2 // P).astype(np.int32)

_mesh = plsc.VectorSubcoreMesh(core_axis_name="c", subcore_axis_name="s")


@functools.partial(
    pl.kernel,
    out_type=jax.ShapeDtypeStruct((NW, 64), jnp.float32),
    mesh=_mesh,
    compiler_params=pltpu.CompilerParams(needs_layout_passes=False),
    scratch_types=[
        pltpu.VMEM((RB, T), jnp.float32),
        pltpu.VMEM((RB, T), jnp.float32),
        pltpu.VMEM((RB, PN), jnp.int32),
        pltpu.VMEM((RB, PN), jnp.int32),
        pltpu.VMEM((RB, P), jnp.int32),
        pltpu.VMEM((RB, P), jnp.int32),
        pltpu.VMEM((RB, P), jnp.int32),
        pltpu.VMEM((RB, P), jnp.int32),
        pltpu.VMEM((MB,), jnp.float32),
        pltpu.VMEM((16,), jnp.float32),
        pltpu.VMEM((64,), jnp.float32),
        pltpu.SemaphoreType.DMA,
        pltpu.SemaphoreType.DMA,
    ],
)
def _hinge_sc(scores_hbm, neg2d_hbm, pos_hbm, lev_hbm, marg_hbm,
              out_hbm,
              sc0, sc1, ng0, ng1, po0, po1, lv0, lv1,
              m_v, marg_v, out_v, sem0, sem1):
    wid = lax.axis_index("s") * NC + lax.axis_index("c")
    pltpu.sync_copy(marg_hbm, marg_v)
    row0w = wid * ROWS_W
    iota16 = lax.iota(jnp.int32, 16)
    bufs = ((sc0, ng0, po0, lv0, sem0), (sc1, ng1, po1, lv1, sem1))
    tail_keep = iota16 >= (16 - NTAILV)

    def issue(slot, bk):
        sc, ng, po, lv, sem = bufs[slot]
        row0 = row0w + bk * RB
        pltpu.async_copy(scores_hbm.at[pl.ds(row0, RB)], sc, sem)
        pltpu.async_copy(neg2d_hbm.at[pl.ds(row0, RB)], ng, sem)
        pltpu.async_copy(pos_hbm.at[pl.ds(row0, RB)], po, sem)
        pltpu.async_copy(lev_hbm.at[pl.ds(row0, RB)], lv, sem)

    def wait_slot(slot):
        sc, ng, po, lv, sem = bufs[slot]
        pltpu.make_async_copy(scores_hbm.at[pl.ds(0, RB)], sc, sem).wait()
        pltpu.make_async_copy(neg2d_hbm.at[pl.ds(0, RB)], ng, sem).wait()
        pltpu.make_async_copy(pos_hbm.at[pl.ds(0, RB)], po, sem).wait()
        pltpu.make_async_copy(lev_hbm.at[pl.ds(0, RB)], lv, sem).wait()

    def compute_block(slot, carry):
        sc, ng, po, lv, _ = bufs[slot]

        @plsc.parallel_loop(0, MB // 16, unroll=4)
        def mloop(i):
            off = i * 16
            j = iota16 + jnp.full((16,), off, jnp.int32)
            mri = j // P
            mpi = j - mri * P
            pos = plsc.load_gather(po, [mri, mpi])
            lev = plsc.load_gather(lv, [mri, mpi])
            safe_pos = jnp.where(pos == -1, 0, pos)
            psc = plsc.load_gather(sc, [mri, safe_pos])
            mg = plsc.load_gather(marg_v, [lev])
            m_v[pl.ds(off, 16)] = mg - psc

        def rowloop(r, cr):
            a = list(cr)
            rsplat = jnp.full((16,), r, jnp.int32)
            rp = r * P
            msp = [plsc.load_gather(m_v, [jnp.full((16,), rp + p, jnp.int32)])
                   for p in range(P)]

            def chunk(k, off, msel, extra_mask, a):
                neg = ng[r, pl.ds(off, 16)]
                msk = neg != -1
                if extra_mask is not None:
                    msk = msk & extra_mask
                safe_neg = jnp.where(msk, neg, 0)
                nsc = plsc.load_gather(sc, [rsplat, safe_neg])
                v = jnp.maximum(msel + nsc, 0.0)
                s = k % 4
                a[s] = a[s] + jnp.where(msk, v, 0.0)
                a[4 + s] = a[4 + s] + jnp.where(msk, 1, 0)
                return a

            for k in range(NFULL):
                off = k * 16
                p_lo = off // N
                p_hi = (off + 15) // N
                if p_lo == p_hi:
                    msel = msp[p_lo]
                else:
                    b = N * p_hi - off
                    msel = jnp.where(iota16 >= b, msp[p_hi], msp[p_lo])
                a = chunk(k, off, msel, None, a)
            a = chunk(NFULL, TAIL0, msp[P - 1], tail_keep, a)
            return tuple(a)

        return lax.fori_loop(0, RB, rowloop, carry)

    issue(0, 0)
    issue(1, 1)
    zf = jnp.zeros((16,), jnp.float32)
    zi = jnp.zeros((16,), jnp.int32)
    acc = (zf, zf, zf, zf, zi, zi, zi, zi)

    def blockpair(t, carry):
        wait_slot(0)
        carry = compute_block(0, carry)

        @pl.when(t < NBLK // 2 - 1)
        def _():
            issue(0, 2 * t + 2)

        wait_slot(1)
        carry = compute_block(1, carry)

        @pl.when(t < NBLK // 2 - 1)
        def _():
            issue(1, 2 * t + 3)

        return carry

    acc = lax.fori_loop(0, NBLK // 2, blockpair, acc)

    out_v[pl.ds(0, 16)] = acc[0] + acc[1]
    out_v[pl.ds(16, 16)] = acc[2] + acc[3]
    out_v[pl.ds(32, 16)] = (acc[4] + acc[5]).astype(jnp.float32)
    out_v[pl.ds(48, 16)] = (acc[6] + acc[7]).astype(jnp.float32)
    pltpu.sync_copy(out_v, out_hbm.at[wid])


def kernel(scores, pos_type_ids, neg_type_ids, levels, margin_ratio, margins):
    marg = (margins * margin_ratio).astype(jnp.float32)
    parts = _hinge_sc(
        scores,
        neg_type_ids.reshape(B, PN),
        pos_type_ids,
        levels,
        marg,
    )
    pr = parts.reshape(NW, 2, 32)
    return pr[:, 0].sum() / pr[:, 1].sum()
